# baseline (speedup 1.0000x reference)
import jax
import jax.numpy as jnp
from jax.experimental import pallas as pl


def _fused_body(x_ref, jid_ref, w1t_ref, b1_ref, w2t_ref, b2_ref,
                w1at_ref, b1a_ref, w2at_ref, b2a_ref,
                vwt_ref, vb_ref, vawt_ref, vab_ref, out_ref):
    x = x_ref[...]
    jid = jid_ref[...]
    B, D = x.shape
    w1t = w1t_ref[...]
    w2t = w2t_ref[...]
    vwt = vwt_ref[...]
    b1 = b1_ref[...]
    b2 = b2_ref[...]
    vb = vb_ref[...]
    J = w1at_ref.shape[0]
    H1 = w1t.shape[1]
    H2 = w2t.shape[1]
    QC = vwt.shape[1]
    f32 = jnp.float32

    def body(j, logits_acc):
        m = (jid == j).astype(f32)
        w1 = w1at_ref[j]
        bb1 = b1a_ref[j]
        w2 = w2at_ref[j]
        bb2 = b2a_ref[j]
        vw = vawt_ref[j]
        vvb = vab_ref[j]
        w1c = w1t + w1
        w2c = w2t + w2
        vwc = vwt + vw
        xm = x * m
        z1 = jnp.maximum(
            jnp.dot(xm, w1c[:D], preferred_element_type=f32)
            + m * (w1c[D:D + 1] + b1 + bb1), 0.0)
        z2 = jnp.maximum(
            jnp.dot(z1, w2c[:H1], preferred_element_type=f32)
            + m * (w2c[H1:H1 + 1] + b2 + bb2), 0.0)
        lg = (jnp.dot(z2, vwc[:H2], preferred_element_type=f32)
              + m * (vwc[H2:H2 + 1] + vb + vvb))
        return logits_acc + lg

    logits = jax.lax.fori_loop(0, J, body, jnp.zeros((B, QC), f32))
    mx = jnp.max(logits, axis=1, keepdims=True)
    e = jnp.exp(logits - mx)
    C = 5
    gi = jax.lax.broadcasted_iota(jnp.int32, (QC, QC), 0) // C
    gj = jax.lax.broadcasted_iota(jnp.int32, (QC, QC), 1) // C
    grp = (gi == gj).astype(f32)
    denom = jnp.dot(e, grp, preferred_element_type=f32)
    out_ref[...] = e / denom


def kernel(x, judge_ids, W1_w, W1_b, W2_w, W2_b, W1a_w, W1a_b, W2a_w, W2a_b,
           V_w, V_b, Va_w, Va_b):
    B, D = x.shape
    J, H1, _ = W1a_w.shape
    H2 = W2a_w.shape[1]
    Q, C, _ = V_w.shape
    QC = Q * C
    jid = judge_ids.astype(jnp.int32).reshape(B, 1)
    w1t = W1_w.T
    w2t = W2_w.T
    vwt = V_w.reshape(QC, H2 + 1).T
    w1at = W1a_w.transpose(0, 2, 1)
    w2at = W2a_w.transpose(0, 2, 1)
    vawt = Va_w.reshape(J, QC, H2 + 1).transpose(0, 2, 1)
    b1 = W1_b.reshape(1, H1)
    b2 = W2_b.reshape(1, H2)
    vb = V_b.reshape(1, QC)
    b1a = W1a_b.reshape(J, 1, H1)
    b2a = W2a_b.reshape(J, 1, H2)
    vab = Va_b.reshape(J, 1, QC)

    out = pl.pallas_call(
        _fused_body,
        out_shape=jax.ShapeDtypeStruct((B, QC), jnp.float32),
    )(x, jid, w1t, b1, w2t, b2, w1at, b1a, w2at, b2a, vwt, vb, vawt, vab)
    return out.reshape(B, Q, C).transpose(1, 0, 2)
